# baseline (speedup 1.0000x reference)
import functools

import jax
import jax.numpy as jnp
from jax import lax
from jax.experimental import pallas as pl
from jax.experimental.pallas import tpu as pltpu
from jax.experimental.pallas import tpu_sc as plsc


def _gather_kernel(B, F, V, D):
    info = plsc.get_sparse_core_info()
    NC, NS, L = info.num_cores, info.num_subcores, info.num_lanes
    NW = NC * NS
    assert B % NW == 0
    bpw = B // NW
    C = 16
    NCHUNK = bpw // C

    mesh = plsc.VectorSubcoreMesh(core_axis_name="c", subcore_axis_name="s")

    @functools.partial(
        pl.kernel,
        mesh=mesh,
        out_type=jax.ShapeDtypeStruct((B, F, D), jnp.float32),
        scratch_types=[
            pltpu.VMEM((F, bpw), jnp.int32),
            pltpu.VMEM((F * bpw + L,), jnp.int32),
        ] + [pltpu.VMEM((C, 8, D), jnp.float32)] * 4
          + [pltpu.VMEM((C, D), jnp.float32)] * 4
          + [pltpu.SemaphoreType.DMA] * 8,
    )
    def k(idx_hbm, tab_hbm, out_hbm, idx_v, sidx_v,
          g0, g1, g2, g3, e0, e1, e2, e3,
          sg0, sg1, sg2, sg3, sw0, sw1, sw2, sw3):
        wid = lax.axis_index("s") * NC + lax.axis_index("c")
        b0 = wid * bpw

        pltpu.sync_copy(idx_hbm.at[:, pl.ds(b0, bpw)], idx_v)

        def stage(i, c):
            f = lax.shift_right_logical(i, 3)
            bb = jnp.bitwise_and(i, 7) * L
            sidx_v[pl.ds(i * L, L)] = idx_v[f, pl.ds(bb, L)]
            return c

        lax.fori_loop(0, (F * bpw) // L, stage, 0)

        grps = (g0, g1, g2, g3)
        exts = (e0, e1, e2, e3)
        sgs = (sg0, sg1, sg2, sg3)
        sws = (sw0, sw1, sw2, sw3)

        def fire(f, q0, grp, sem):
            def go(i, c):
                w = sidx_v[pl.ds(f * bpw + q0 + i * L, L)]
                for kk in range(L):
                    vg = lax.shift_right_logical(w[kk], 3)
                    pltpu.async_copy(
                        tab_hbm.at[f, vg, :, :], grp.at[i * L + kk], sem
                    )
                return c

            lax.fori_loop(0, C // L, go, 0)

        def drain_g(grp, sem):
            pltpu.make_async_copy(
                tab_hbm.at[0, pl.ds(0, C), :, :], grp, sem
            ).wait()

        def extract(f, q0, grp, ext):
            def go(i, c):
                w = sidx_v[pl.ds(f * bpw + q0 + i * L, L)]
                for kk in range(L):
                    j = i * L + kk
                    s = jnp.bitwise_and(w[kk], 7)
                    for h in range(D // L):
                        ext[j, pl.ds(h * L, L)] = grp[j, s, pl.ds(h * L, L)]
                return c

            lax.fori_loop(0, C // L, go, 0)

        def out_write(f, q0, ext, sem):
            pltpu.async_copy(ext, out_hbm.at[pl.ds(b0 + q0, C), f, :], sem)

        def drain_w(ext, sem):
            pltpu.make_async_copy(
                ext, out_hbm.at[pl.ds(b0, C), 0, :], sem
            ).wait()

        NB = 4

        def field(fi, carry):
            for q in range(NB - 1):
                fire(fi, q * C, grps[q], sgs[q])
            for q in range(NCHUNK):
                p = q % NB
                if q + NB - 1 < NCHUNK:
                    qq = q + NB - 1
                    fire(fi, qq * C, grps[qq % NB], sgs[qq % NB])
                drain_g(grps[p], sgs[p])
                if q >= NB:
                    drain_w(exts[p], sws[p])
                else:
                    @pl.when(fi > 0)
                    def _():
                        drain_w(exts[p], sws[p])
                extract(fi, q * C, grps[p], exts[p])
                out_write(fi, q * C, exts[p], sws[p])
            return carry

        lax.fori_loop(0, F, field, 0)
        for q in range(NB):
            drain_w(exts[q], sws[q])

    return k


def kernel(inputs, tables):
    B, F = inputs.shape
    _, V, D = tables.shape
    idx_t = inputs.T
    tab4 = tables.reshape(F, V // 8, 8, D)
    return _gather_kernel(B, F, V, D)(idx_t, tab4)
